# baseline (speedup 1.0000x reference)
import functools

import jax
import jax.numpy as jnp
from jax import lax
from jax.experimental import pallas as pl
from jax.experimental.pallas import tpu as pltpu
from jax.experimental.pallas import tpu_sc as plsc

_SEQ = 200
_BATCH = 4096
_EMBED = 64

_NC, _NS = 2, 16
_NW = _NC * _NS
_G = _BATCH // _NW
_NBUF = 2
_L = 16

_mesh = plsc.VectorSubcoreMesh(core_axis_name="c", subcore_axis_name="s")


@functools.partial(
    pl.kernel,
    mesh=_mesh,
    out_type=jax.ShapeDtypeStruct((_SEQ, 8, _NW, 8, _G), jnp.float32),
    scratch_types=[
        pltpu.VMEM((_SEQ, _G), jnp.int32),
        pltpu.VMEM((_SEQ, _G), jnp.int32),
        pltpu.VMEM((_NBUF, _G, 2 * _EMBED), jnp.float32),
        pltpu.VMEM((_NBUF, 8, 8, _G), jnp.float32),
        pltpu.SemaphoreType.DMA,
        pltpu.SemaphoreType.DMA,
        pltpu.SemaphoreType.DMA,
        pltpu.SemaphoreType.DMA,
    ],
    compiler_params=pltpu.CompilerParams(
        use_tc_tiling_on_sc=False, needs_layout_passes=False),
)
def _embed_gather(idx_hbm, table_hbm, out_hbm, idx_v, idxh_v, praw_v,
                  trow_v, g0, g1, w0, w1):
    gsem = [g0, g1]
    wsem = [w0, w1]
    wid = lax.axis_index("s") * _NC + lax.axis_index("c")
    col0 = wid * _G

    def gstart(ci, b):
        pltpu.async_copy(
            table_hbm.at[idxh_v.at[ci]], praw_v.at[b], gsem[b])

    def gwait(ci, b):
        pltpu.make_async_copy(
            table_hbm.at[idxh_v.at[ci]], praw_v.at[b], gsem[b]).wait()

    def wb(ci, b, sem):
        return pltpu.make_async_copy(
            trow_v.at[b], out_hbm.at[ci, pl.ds(0, 8), wid], sem)

    pltpu.sync_copy(idx_hbm.at[:, pl.ds(col0, _G)], idx_v)

    def halve(l, carry):
        for k in range(_G // _L):
            v = idx_v[l, pl.ds(k * _L, _L)]
            idxh_v[l, pl.ds(k * _L, _L)] = lax.shift_right_logical(v, 1)
        return carry

    lax.fori_loop(0, _SEQ, halve, 0)

    iota = lax.iota(jnp.int32, _L)

    def transpose(ci, b):
        praw = praw_v.at[b]
        hvs = []
        for k in range(_G // _L):
            v = idx_v[ci, pl.ds(k * _L, _L)]
            hvs.append(lax.shift_left(lax.bitwise_and(v, 1), 6))
        for tr in range(8):
            for r in range(8):
                for k in range(_G // _L):
                    rows = iota + (k * _L)
                    cols = hvs[k] + (tr * 8 + r)
                    vals = plsc.load_gather(praw, [rows, cols])
                    trow_v[b, tr, r, pl.ds(k * _L, _L)] = vals

    for b in range(_NBUF):
        gstart(b, b)

    def group(g, carry):
        ci0 = g * _NBUF
        for b in range(_NBUF):
            ci = ci0 + b
            gwait(ci, b)

            @pl.when(g > 0)
            def _():
                wb(ci - _NBUF, b, wsem[b]).wait()

            transpose(ci, b)

            @pl.when(g < _SEQ // _NBUF - 1)
            def _():
                gstart(ci + _NBUF, b)

            wb(ci, b, wsem[b]).start()
        return carry

    lax.fori_loop(0, _SEQ // _NBUF, group, 0)

    ci0 = _SEQ - _NBUF
    for b in range(_NBUF):
        wb(ci0 + b, b, wsem[b]).wait()


def kernel(indices, table):
    table2 = table.reshape(500000, 2 * _EMBED)
    out5 = _embed_gather(indices.astype(jnp.int32), table2)
    return jnp.transpose(out5, (0, 2, 4, 1, 3)).reshape(_SEQ, _BATCH, _EMBED)


# speedup vs baseline: 1.9839x; 1.9839x over previous
import functools

import jax
import jax.numpy as jnp
from jax import lax
from jax.experimental import pallas as pl
from jax.experimental.pallas import tpu as pltpu
from jax.experimental.pallas import tpu_sc as plsc

_SEQ = 200
_BATCH = 4096
_EMBED = 64
_B = _SEQ * _BATCH

_NC, _NS = 2, 16
_NW = _NC * _NS
_G = _BATCH // _NW
_NBUF = 2
_P = 2 * _EMBED

_mesh = plsc.VectorSubcoreMesh(core_axis_name="c", subcore_axis_name="s")


@functools.partial(
    pl.kernel,
    mesh=_mesh,
    out_type=jax.ShapeDtypeStruct((_B, _P), jnp.float32),
    scratch_types=[
        pltpu.VMEM((_SEQ, _G), jnp.int32),
        pltpu.VMEM((_NBUF, _G, _P), jnp.float32),
        pltpu.SemaphoreType.DMA,
        pltpu.SemaphoreType.DMA,
        pltpu.SemaphoreType.DMA,
        pltpu.SemaphoreType.DMA,
    ],
    compiler_params=pltpu.CompilerParams(use_tc_tiling_on_sc=False),
)
def _embed_gather(idx_hbm, table_hbm, out_hbm, idx_v, praw_v,
                  g0, g1, w0, w1):
    gsem = [g0, g1]
    wsem = [w0, w1]
    wid = lax.axis_index("s") * _NC + lax.axis_index("c")
    col0 = wid * _G

    def gstart(ci, b):
        pltpu.async_copy(
            table_hbm.at[idx_v.at[ci]], praw_v.at[b], gsem[b])

    def gwait(ci, b):
        pltpu.make_async_copy(
            table_hbm.at[idx_v.at[ci]], praw_v.at[b], gsem[b]).wait()

    def wb(ci, b, sem):
        return pltpu.make_async_copy(
            praw_v.at[b], out_hbm.at[pl.ds(ci * _BATCH + col0, _G)], sem)

    pltpu.sync_copy(idx_hbm.at[:, pl.ds(col0, _G)], idx_v)

    for b in range(_NBUF):
        gstart(b, b)

    def group(g, carry):
        ci0 = g * _NBUF
        for b in range(_NBUF):
            ci = ci0 + b
            gwait(ci, b)

            @pl.when(g > 0)
            def _():
                wb(ci - _NBUF, b, wsem[b]).wait()

            @pl.when(g < _SEQ // _NBUF - 1)
            def _():
                gstart(ci + _NBUF, b)

            wb(ci, b, wsem[b]).start()
        return carry

    lax.fori_loop(0, _SEQ // _NBUF, group, 0)

    ci0 = _SEQ - _NBUF
    for b in range(_NBUF):
        wb(ci0 + b, b, wsem[b]).wait()


def kernel(indices, table):
    tpad = jnp.pad(table, ((0, 0), (0, _P - _EMBED)))
    outp = _embed_gather(indices.astype(jnp.int32), tpad)
    return outp[:, :_EMBED].reshape(_SEQ, _BATCH, _EMBED)


# speedup vs baseline: 1.9862x; 1.0012x over previous
import functools

import jax
import jax.numpy as jnp
from jax import lax
from jax.experimental import pallas as pl
from jax.experimental.pallas import tpu as pltpu
from jax.experimental.pallas import tpu_sc as plsc

_SEQ = 200
_BATCH = 4096
_EMBED = 64
_B = _SEQ * _BATCH

_NC, _NS = 2, 16
_NW = _NC * _NS
_G = _BATCH // _NW
_NBUF = 4
_P = 2 * _EMBED

_mesh = plsc.VectorSubcoreMesh(core_axis_name="c", subcore_axis_name="s")


@functools.partial(
    pl.kernel,
    mesh=_mesh,
    out_type=jax.ShapeDtypeStruct((_B, _P), jnp.float32),
    scratch_types=[
        pltpu.VMEM((_SEQ, _G), jnp.int32),
        pltpu.VMEM((_NBUF, _G, _P), jnp.float32),
        pltpu.SemaphoreType.DMA,
        pltpu.SemaphoreType.DMA,
        pltpu.SemaphoreType.DMA,
        pltpu.SemaphoreType.DMA,
        pltpu.SemaphoreType.DMA,
        pltpu.SemaphoreType.DMA,
        pltpu.SemaphoreType.DMA,
        pltpu.SemaphoreType.DMA,
    ],
    compiler_params=pltpu.CompilerParams(use_tc_tiling_on_sc=False),
)
def _embed_gather(idx_hbm, table_hbm, out_hbm, idx_v, praw_v,
                  g0, g1, g2, g3, w0, w1, w2, w3):
    gsem = [g0, g1, g2, g3]
    wsem = [w0, w1, w2, w3]
    wid = lax.axis_index("s") * _NC + lax.axis_index("c")
    col0 = wid * _G

    def gstart(ci, b):
        pltpu.async_copy(
            table_hbm.at[idx_v.at[ci]], praw_v.at[b], gsem[b])

    def gwait(ci, b):
        pltpu.make_async_copy(
            table_hbm.at[idx_v.at[ci]], praw_v.at[b], gsem[b]).wait()

    def wb(ci, b, sem):
        return pltpu.make_async_copy(
            praw_v.at[b], out_hbm.at[pl.ds(ci * _BATCH + col0, _G)], sem)

    pltpu.sync_copy(idx_hbm.at[:, pl.ds(col0, _G)], idx_v)

    gstart(0, 0)
    gstart(1, 1)

    def group(g, carry):
        ci0 = g * _NBUF
        for q in range(_NBUF):
            ci = ci0 + q
            gwait(ci, q)

            @pl.when(ci >= 2)
            def _():
                wb(ci - 2, (q + 2) % _NBUF, wsem[(q + 2) % _NBUF]).wait()

            @pl.when(ci + 2 < _SEQ)
            def _():
                gstart(ci + 2, (q + 2) % _NBUF)

            wb(ci, q, wsem[q]).start()
        return carry

    lax.fori_loop(0, _SEQ // _NBUF, group, 0)

    for ci in range(_SEQ - 2, _SEQ):
        wb(ci, ci % _NBUF, wsem[ci % _NBUF]).wait()


def kernel(indices, table):
    tpad = jnp.pad(table, ((0, 0), (0, _P - _EMBED)))
    outp = _embed_gather(indices.astype(jnp.int32), tpad)
    return outp[:, :_EMBED].reshape(_SEQ, _BATCH, _EMBED)
